# baseline (speedup 1.0000x reference)
import functools

import jax
import jax.numpy as jnp
from jax import lax
from jax.experimental import pallas as pl
from jax.experimental.pallas import tpu as pltpu
from jax.experimental.pallas import tpu_sc as plsc

B = 16384
V = 1000000
D = 16
L = 16


def _scalar(vec, j):
    return jnp.reshape(lax.slice(vec, (j,), (j + 1,)), ())


@functools.cache
def _build(num_cores, num_subcores):
    nw = num_cores * num_subcores
    b_per_w = B // nw
    groups = b_per_w // L
    mesh = plsc.VectorSubcoreMesh(
        core_axis_name="c", subcore_axis_name="s",
        num_cores=num_cores, num_subcores=num_subcores)

    @functools.partial(
        pl.kernel,
        out_type=jax.ShapeDtypeStruct((B,), jnp.float32),
        mesh=mesh,
        scratch_types=[
            pltpu.VMEM((b_per_w,), jnp.int32),
            pltpu.VMEM((b_per_w,), jnp.int32),
            pltpu.VMEM((L, D, 128), jnp.float32),
            pltpu.VMEM((L, D, 128), jnp.float32),
            pltpu.VMEM((D * b_per_w,), jnp.float32),
            pltpu.VMEM((b_per_w,), jnp.float32),
            pltpu.SemaphoreType.DMA,
            pltpu.SemaphoreType.DMA,
            pltpu.SemaphoreType.DMA,
            pltpu.SemaphoreType.DMA,
        ],
        compiler_params=pltpu.CompilerParams(
            needs_layout_passes=False, use_tc_tiling_on_sc=True),
    )
    def mf_kernel(uids_hbm, iids_hbm, utab_hbm, itab_hbm, out_hbm,
                  uidx_v, iidx_v, uring_v, iring_v, prod_v, out_v,
                  sem_u0, sem_i0, sem_u1, sem_i1):
        wid = lax.axis_index("s") * num_cores + lax.axis_index("c")
        base = wid * b_per_w
        pltpu.sync_copy(uids_hbm.at[pl.ds(base, b_per_w)], uidx_v)
        pltpu.sync_copy(iids_hbm.at[pl.ds(base, b_per_w)], iidx_v)

        lanes = lax.iota(jnp.int32, L)
        half = L // 2

        def issue_half(g, h, sem_u, sem_i):
            o = g * L
            uvec = uidx_v[pl.ds(o, L)]
            ivec = iidx_v[pl.ds(o, L)]
            ublk = (uvec >> 7) << 7
            iblk = (ivec >> 7) << 7
            for j in range(h * half, h * half + half):
                cu = pl.multiple_of(_scalar(ublk, j), 128)
                ci = pl.multiple_of(_scalar(iblk, j), 128)
                pltpu.async_copy(utab_hbm.at[pl.ds(0, 8), pl.ds(cu, 128)],
                                 uring_v.at[j, pl.ds(0, 8)], sem_u)
                pltpu.async_copy(utab_hbm.at[pl.ds(8, 8), pl.ds(cu, 128)],
                                 uring_v.at[j, pl.ds(8, 8)], sem_u)
                pltpu.async_copy(itab_hbm.at[pl.ds(0, 8), pl.ds(ci, 128)],
                                 iring_v.at[j, pl.ds(0, 8)], sem_i)
                pltpu.async_copy(itab_hbm.at[pl.ds(8, 8), pl.ds(ci, 128)],
                                 iring_v.at[j, pl.ds(8, 8)], sem_i)

        def compute_half(g, h, sem_u, sem_i):
            o = g * L
            for _ in range(half):
                pltpu.make_async_copy(utab_hbm.at[:, pl.ds(0, 128)],
                                      uring_v.at[0], sem_u).wait()
                pltpu.make_async_copy(itab_hbm.at[:, pl.ds(0, 128)],
                                      iring_v.at[0], sem_i).wait()
            uvec = uidx_v[pl.ds(o, L)] & 127
            ivec = iidx_v[pl.ds(o, L)] & 127
            for j in range(h * half, h * half + half):
                wu = jnp.full((L,), _scalar(uvec, j), jnp.int32)
                wi = jnp.full((L,), _scalar(ivec, j), jnp.int32)
                uv = plsc.load_gather(uring_v.at[j], [lanes, wu])
                iv = plsc.load_gather(iring_v.at[j], [lanes, wi])
                plsc.store_scatter(prod_v, [lanes * b_per_w + (o + j)],
                                   uv * iv)

        issue_half(0, 0, sem_u0, sem_i0)
        issue_half(0, 1, sem_u1, sem_i1)

        def body(g, carry):
            compute_half(g, 0, sem_u0, sem_i0)

            @pl.when(g < groups - 1)
            def _():
                issue_half(g + 1, 0, sem_u0, sem_i0)
            compute_half(g, 1, sem_u1, sem_i1)

            @pl.when(g < groups - 1)
            def _():
                issue_half(g + 1, 1, sem_u1, sem_i1)
            return carry

        lax.fori_loop(0, groups, body, 0)

        def red_group(g, carry):
            o = g * L
            acc = jnp.zeros((L,), jnp.float32)
            for d in range(D):
                acc = acc + prod_v[pl.ds(d * b_per_w + o, L)]
            out_v[pl.ds(o, L)] = acc
            return carry
        lax.fori_loop(0, groups, red_group, 0)

        pltpu.sync_copy(out_v, out_hbm.at[pl.ds(base, b_per_w)])

    return mf_kernel


def kernel(user_ids, item_ids, user_table, item_table):
    try:
        info = plsc.get_sparse_core_info()
        nc, ns = info.num_cores, info.num_subcores
    except Exception:
        nc, ns = 2, 16
    return _build(nc, ns)(user_ids, item_ids, user_table.T, item_table.T)


# speedup vs baseline: 1.0143x; 1.0143x over previous
import functools

import jax
import jax.numpy as jnp
from jax import lax
from jax.experimental import pallas as pl
from jax.experimental.pallas import tpu as pltpu
from jax.experimental.pallas import tpu_sc as plsc

B = 16384
V = 1000000
D = 16
L = 16
U = 8


def _scalar(vec, j):
    return jnp.reshape(lax.slice(vec, (j,), (j + 1,)), ())


@functools.cache
def _build(num_cores, num_subcores):
    nw = num_cores * num_subcores
    b_per_w = B // nw
    units = b_per_w // U
    mesh = plsc.VectorSubcoreMesh(
        core_axis_name="c", subcore_axis_name="s",
        num_cores=num_cores, num_subcores=num_subcores)

    @functools.partial(
        pl.kernel,
        out_type=jax.ShapeDtypeStruct((B,), jnp.float32),
        mesh=mesh,
        scratch_types=[
            pltpu.VMEM((b_per_w + U,), jnp.int32),
            pltpu.VMEM((b_per_w + U,), jnp.int32),
            pltpu.VMEM((3, U, D, 128), jnp.float32),
            pltpu.VMEM((3, U, D, 128), jnp.float32),
            pltpu.VMEM((D * L,), jnp.float32),
            pltpu.VMEM((b_per_w,), jnp.float32),
            pltpu.SemaphoreType.DMA,
            pltpu.SemaphoreType.DMA,
            pltpu.SemaphoreType.DMA,
            pltpu.SemaphoreType.DMA,
            pltpu.SemaphoreType.DMA,
            pltpu.SemaphoreType.DMA,
        ],
        compiler_params=pltpu.CompilerParams(
            needs_layout_passes=False, use_tc_tiling_on_sc=True),
    )
    def mf_kernel(uids_hbm, iids_hbm, utab_hbm, itab_hbm, out_hbm,
                  uidx_v, iidx_v, ubuf_v, ibuf_v, prod_v, out_v,
                  su0, si0, su1, si1, su2, si2):
        wid = lax.axis_index("s") * num_cores + lax.axis_index("c")
        base = wid * b_per_w
        pltpu.sync_copy(uids_hbm.at[pl.ds(base, b_per_w)],
                        uidx_v.at[pl.ds(0, b_per_w)])
        pltpu.sync_copy(iids_hbm.at[pl.ds(base, b_per_w)],
                        iidx_v.at[pl.ds(0, b_per_w)])

        lanes = lax.iota(jnp.int32, L)
        sems = ((su0, si0), (su1, si1), (su2, si2))

        def issue_unit(u, b):
            sem_u, sem_i = sems[b]
            uvec = uidx_v[pl.ds(u * U, L)]
            ivec = iidx_v[pl.ds(u * U, L)]
            ublk = (uvec >> 7) << 7
            iblk = (ivec >> 7) << 7
            for j in range(U):
                cu = pl.multiple_of(_scalar(ublk, j), 128)
                ci = pl.multiple_of(_scalar(iblk, j), 128)
                pltpu.async_copy(utab_hbm.at[:, pl.ds(cu, 128)],
                                 ubuf_v.at[b, j], sem_u)
                pltpu.async_copy(itab_hbm.at[:, pl.ds(ci, 128)],
                                 ibuf_v.at[b, j], sem_i)

        def compute_unit(u, b, parity):
            sem_u, sem_i = sems[b]
            for _ in range(U):
                pltpu.make_async_copy(utab_hbm.at[:, pl.ds(0, 128)],
                                      ubuf_v.at[b, 0], sem_u).wait()
                pltpu.make_async_copy(itab_hbm.at[:, pl.ds(0, 128)],
                                      ibuf_v.at[b, 0], sem_i).wait()
            uvec = uidx_v[pl.ds(u * U, L)] & 127
            ivec = iidx_v[pl.ds(u * U, L)] & 127
            for j in range(U):
                wu = jnp.full((L,), _scalar(uvec, j), jnp.int32)
                wi = jnp.full((L,), _scalar(ivec, j), jnp.int32)
                uv = plsc.load_gather(ubuf_v.at[b, j], [lanes, wu])
                iv = plsc.load_gather(ibuf_v.at[b, j], [lanes, wi])
                plsc.store_scatter(prod_v, [lanes * L + (parity * U + j)],
                                   uv * iv)
            @pl.when(parity == 1)
            def _():
                acc = jnp.zeros((L,), jnp.float32)
                for d in range(D):
                    acc2 = acc + prod_v[pl.ds(d * L, L)]
                    acc = acc2
                out_v[pl.ds((u - 1) * U, L)] = acc

        issue_unit(0, 0)
        issue_unit(1, 1)

        def body(k, carry):
            u = 3 * k
            compute_unit(u, 0, u % 2)

            @pl.when(u + 2 < units)
            def _():
                issue_unit(u + 2, 2)
            compute_unit(u + 1, 1, (u + 1) % 2)

            @pl.when(u + 3 < units)
            def _():
                issue_unit(u + 3, 0)
            compute_unit(u + 2, 2, (u + 2) % 2)

            @pl.when(u + 4 < units)
            def _():
                issue_unit(u + 4, 1)
            return carry

        lax.fori_loop(0, units // 3, body, 0)
        compute_unit(units - 1, (units - 1) % 3, (units - 1) % 2)

        pltpu.sync_copy(out_v, out_hbm.at[pl.ds(base, b_per_w)])

    return mf_kernel


def kernel(user_ids, item_ids, user_table, item_table):
    try:
        info = plsc.get_sparse_core_info()
        nc, ns = info.num_cores, info.num_subcores
    except Exception:
        nc, ns = 2, 16
    return _build(nc, ns)(user_ids, item_ids, user_table.T, item_table.T)
